# Initial kernel scaffold; baseline (speedup 1.0000x reference)
import jax
import jax.numpy as jnp
from jax.experimental import pallas as pl


def kernel(x, edge_index, edge_weight, W, b):
    raise NotImplementedError("write your pallas kernel here")


# speedup vs baseline: 11.4627x; 11.4627x over previous
import functools

import jax
import jax.numpy as jnp
from jax import lax
from jax.experimental import pallas as pl
from jax.experimental.pallas import tpu as pltpu
from jax.experimental.pallas import tpu_sc as plsc

NC = 2
NS = 16
L = 16
CHUNK = 80


def _sc_deg_kernel(n_pad, chunks_per_tile, et):
    mesh = plsc.VectorSubcoreMesh(core_axis_name="c", subcore_axis_name="s")
    npt = n_pad // NS

    @functools.partial(
        pl.kernel,
        mesh=mesh,
        compiler_params=pltpu.CompilerParams(use_tc_tiling_on_sc=False),
        out_type=jax.ShapeDtypeStruct((NC, n_pad), jnp.float32),
        scratch_types=[
            pltpu.VMEM_SHARED((n_pad,), jnp.float32),
            pltpu.VMEM((CHUNK,), jnp.int32),
            pltpu.VMEM((CHUNK,), jnp.float32),
            pltpu.VMEM((n_pad // NS,), jnp.float32),
        ],
    )
    def k(col_hbm, ew_hbm, out_hbm, shared_deg, colv, ewv, zbuf):
        cid = lax.axis_index("c")
        sid = lax.axis_index("s")
        w = cid * NS + sid

        def zb(i, _):
            zbuf[pl.ds(i * L, L)] = jnp.zeros((L,), jnp.float32)
            return 0

        lax.fori_loop(0, npt // L, zb, 0)
        pltpu.sync_copy(zbuf, shared_deg.at[pl.ds(sid * npt, npt)])
        plsc.subcore_barrier()

        def body(c, _):
            base = w * et + c * CHUNK
            pltpu.sync_copy(col_hbm.at[pl.ds(base, CHUNK)], colv)
            pltpu.sync_copy(ew_hbm.at[pl.ds(base, CHUNK)], ewv)
            pltpu.sync_copy(ewv, shared_deg.at[colv], add=True)
            return 0

        lax.fori_loop(0, chunks_per_tile, body, 0)
        plsc.subcore_barrier()
        pltpu.sync_copy(
            shared_deg.at[pl.ds(sid * npt, npt)],
            out_hbm.at[cid, pl.ds(sid * npt, npt)],
        )

    return k


def _sc_msg_kernel(n_pad, chunks_per_tile, et, c_feat):
    mesh = plsc.VectorSubcoreMesh(core_axis_name="c", subcore_axis_name="s")
    npt = n_pad // NS

    @functools.partial(
        pl.kernel,
        mesh=mesh,
        compiler_params=pltpu.CompilerParams(use_tc_tiling_on_sc=False),
        out_type=jax.ShapeDtypeStruct((NC, n_pad, c_feat), jnp.float32),
        scratch_types=[
            pltpu.VMEM_SHARED((n_pad, c_feat), jnp.float32),
            pltpu.VMEM((CHUNK,), jnp.int32),
            pltpu.VMEM((CHUNK,), jnp.int32),
            pltpu.VMEM((CHUNK,), jnp.float32),
            pltpu.VMEM((CHUNK, c_feat), jnp.float32),
            pltpu.VMEM((CHUNK,), jnp.float32),
            pltpu.VMEM((CHUNK,), jnp.float32),
            pltpu.VMEM((n_pad // NS, c_feat), jnp.float32),
        ],
    )
    def k(row_hbm, col_hbm, ew_hbm, h_hbm, dinv_hbm, out_hbm,
          shared_out, rowv, colv, ewv, hrows, drv, dcv, zbuf):
        cid = lax.axis_index("c")
        sid = lax.axis_index("s")
        w = cid * NS + sid

        def zb(i, _):
            zbuf[i, :] = jnp.zeros((L,), jnp.float32)
            return 0

        lax.fori_loop(0, npt, zb, 0)
        pltpu.sync_copy(zbuf, shared_out.at[pl.ds(sid * npt, npt), :])
        plsc.subcore_barrier()

        def body(c, _):
            base = w * et + c * CHUNK
            pltpu.sync_copy(row_hbm.at[pl.ds(base, CHUNK)], rowv)
            pltpu.sync_copy(col_hbm.at[pl.ds(base, CHUNK)], colv)
            pltpu.sync_copy(ew_hbm.at[pl.ds(base, CHUNK)], ewv)
            pltpu.sync_copy(h_hbm.at[rowv], hrows)
            pltpu.sync_copy(dinv_hbm.at[rowv], drv)
            pltpu.sync_copy(dinv_hbm.at[colv], dcv)
            for kk in range(CHUNK // L):
                s = pl.ds(kk * L, L)
                nv = drv[s] * ewv[s] * dcv[s]
                for j in range(L):
                    e = kk * L + j
                    hrows[e, :] = hrows[e, :] * nv[j]
            pltpu.sync_copy(hrows, shared_out.at[colv], add=True)
            return 0

        lax.fori_loop(0, chunks_per_tile, body, 0)
        plsc.subcore_barrier()
        pltpu.sync_copy(
            shared_out.at[pl.ds(sid * npt, npt), :],
            out_hbm.at[cid, pl.ds(sid * npt, npt), :],
        )

    return k


def _tc_h_dinv(x, W, d0, d1):
    n, f = x.shape
    c = W.shape[1]

    def body(x_ref, w_ref, d0_ref, d1_ref, h_ref, dinv_ref):
        h_ref[...] = jnp.dot(x_ref[...], w_ref[...],
                             preferred_element_type=jnp.float32)
        d = d0_ref[...] + d1_ref[...]
        dinv_ref[...] = jnp.where(d > 0, lax.rsqrt(jnp.where(d > 0, d, 1.0)), 0.0)

    return pl.pallas_call(
        body,
        out_shape=(
            jax.ShapeDtypeStruct((n, c), jnp.float32),
            jax.ShapeDtypeStruct(d0.shape, jnp.float32),
        ),
    )(x, W, d0, d1)


def _tc_finish(p0, p1, b2d):
    def body(p0_ref, p1_ref, b_ref, o_ref):
        o_ref[...] = jax.nn.sigmoid(p0_ref[...] + p1_ref[...] + b_ref[...])

    return pl.pallas_call(
        body,
        out_shape=jax.ShapeDtypeStruct(p0.shape, jnp.float32),
    )(p0, p1, b2d)


def kernel(x, edge_index, edge_weight, W, b):
    n, f = x.shape
    c = W.shape[1]
    e = edge_index.shape[1]

    n_pad = ((n + NC * 128 - 1) // (NC * 128)) * NC * 128

    loop = jnp.arange(n, dtype=edge_index.dtype)
    e_real = e + n
    grp = NC * NS * CHUNK
    e_pad = ((e_real + grp - 1) // grp) * grp
    pad = e_pad - e_real
    row = jnp.concatenate([edge_index[0], loop,
                           jnp.zeros((pad,), edge_index.dtype)])
    col = jnp.concatenate([edge_index[1], loop,
                           jnp.zeros((pad,), edge_index.dtype)])
    ew = jnp.concatenate([edge_weight, jnp.ones((n,), edge_weight.dtype),
                          jnp.zeros((pad,), edge_weight.dtype)])
    et = e_pad // (NC * NS)
    chunks_per_tile = et // CHUNK

    deg_part = _sc_deg_kernel(n_pad, chunks_per_tile, et)(col, ew)

    d0 = deg_part[0].reshape(n_pad // 128, 128)
    d1 = deg_part[1].reshape(n_pad // 128, 128)
    h, dinv2d = _tc_h_dinv(x, W, d0, d1)
    dinv = dinv2d.reshape(n_pad)

    out_part = _sc_msg_kernel(n_pad, chunks_per_tile, et, c)(
        row, col, ew, h, dinv)

    out = _tc_finish(out_part[0], out_part[1], b.reshape(1, c))
    return out[:n]
